# baseline (speedup 1.0000x reference)
import jax
import jax.numpy as jnp
from jax.experimental import pallas as pl
from jax.experimental.pallas import tpu as pltpu

_BN = 512


def _vq_argmin_kernel(x_ref, c_ref, o_ref):
    x = x_ref[...]
    k = c_ref.shape[0]
    bn = x.shape[0]
    x2 = jnp.sum(x * x, axis=1)[None, :]
    c = c_ref[...]
    c2 = jnp.sum(c * c, axis=1)[:, None]
    ct = jax.lax.dot_general(
        c, x * (-2.0), (((1,), (1,)), ((), ())),
        preferred_element_type=jnp.float32,
    )
    run_v = run_i = None
    for j in range(k // 8):
        s2 = x2 + c2[j * 8:(j + 1) * 8, :]
        d2 = s2 + ct[j * 8:(j + 1) * 8, :]
        dist = jnp.sqrt(jnp.maximum(d2, 0.0))
        if run_v is None:
            run_v = dist
            run_i = jnp.zeros(dist.shape, jnp.int32)
        else:
            lt = dist < run_v
            run_v = jnp.where(lt, dist, run_v)
            run_i = jnp.where(lt, jnp.int32(j), run_i)
    srow = jax.lax.broadcasted_iota(jnp.int32, (8, bn), 0)
    kfull = run_i * 8 + srow
    m = jnp.min(run_v, axis=0, keepdims=True)
    cand = jnp.where(run_v == m, kfull, jnp.int32(k))
    o_ref[...] = jnp.min(cand, axis=0)


def kernel(x, centroids):
    n, d = x.shape
    k, _ = centroids.shape
    grid = (n // _BN,)
    return pl.pallas_call(
        _vq_argmin_kernel,
        grid=grid,
        in_specs=[
            pl.BlockSpec((_BN, d), lambda i: (i, 0)),
            pl.BlockSpec((k, d), lambda i: (0, 0)),
        ],
        out_specs=pl.BlockSpec((_BN,), lambda i: (i,)),
        out_shape=jax.ShapeDtypeStruct((n,), jnp.int32),
        compiler_params=pltpu.CompilerParams(
            dimension_semantics=("parallel",),
        ),
    )(x, centroids)


# speedup vs baseline: 53.4251x; 53.4251x over previous
import jax
import jax.numpy as jnp
from jax.experimental import pallas as pl
from jax.experimental.pallas import tpu as pltpu

_BN = 256
_CH = 128


def _vq_argmin_kernel(x_ref, c_ref, o_ref):
    x = x_ref[...]
    k = c_ref.shape[0]
    bn = x.shape[0]
    xm2 = x * (-2.0)
    x2 = jnp.sum(x * x, axis=1, keepdims=True)
    x2b = jnp.broadcast_to(x2, (bn, _CH))
    run_v = run_i = None
    for j in range(k // _CH):
        cj = c_ref[pl.ds(j * _CH, _CH), :]
        ct = jax.lax.dot_general(
            xm2, cj, (((1,), (1,)), ((), ())),
            preferred_element_type=jnp.float32,
        )
        c2 = jnp.sum(cj * cj, axis=1)[None, :]
        s2 = x2b + c2
        d2 = s2 + ct
        dist = jnp.sqrt(jnp.maximum(d2, 0.0))
        if run_v is None:
            run_v = dist
            run_i = jnp.zeros(dist.shape, jnp.int32)
        else:
            lt = dist < run_v
            run_v = jnp.where(lt, dist, run_v)
            run_i = jnp.where(lt, jnp.int32(j), run_i)
    lane = jax.lax.broadcasted_iota(jnp.int32, (bn, _CH), 1)
    kfull = run_i * _CH + lane
    m = jnp.min(run_v, axis=1, keepdims=True)
    cand = jnp.where(run_v == m, kfull, jnp.int32(k))
    o_ref[...] = jnp.min(cand, axis=1)


def kernel(x, centroids):
    n, d = x.shape
    k, _ = centroids.shape
    grid = (n // _BN,)
    return pl.pallas_call(
        _vq_argmin_kernel,
        grid=grid,
        in_specs=[
            pl.BlockSpec((_BN, d), lambda i: (i, 0)),
            pl.BlockSpec((k, d), lambda i: (0, 0)),
        ],
        out_specs=pl.BlockSpec((_BN,), lambda i: (i,)),
        out_shape=jax.ShapeDtypeStruct((n,), jnp.int32),
        compiler_params=pltpu.CompilerParams(
            dimension_semantics=("parallel",),
        ),
    )(x, centroids)


# speedup vs baseline: 64.2283x; 1.2022x over previous
import jax
import jax.numpy as jnp
from jax.experimental import pallas as pl
from jax.experimental.pallas import tpu as pltpu

_BN = 512
_CH = 128


def _vq_argmin_kernel(x_ref, c_ref, o_ref):
    x = x_ref[...]
    k = c_ref.shape[0]
    bn = x.shape[0]
    xm2 = x * (-2.0)
    x2 = jnp.sum(x * x, axis=1, keepdims=True)
    x2b = jnp.broadcast_to(x2, (bn, _CH))
    run_v = run_i = None
    for j in range(k // _CH):
        cj = c_ref[pl.ds(j * _CH, _CH), :]
        ct = jax.lax.dot_general(
            xm2, cj, (((1,), (1,)), ((), ())),
            preferred_element_type=jnp.float32,
        )
        c2 = jnp.sum(cj * cj, axis=1)[None, :]
        s2 = x2b + c2
        d2 = s2 + ct
        dist = jnp.sqrt(jnp.maximum(d2, 0.0))
        if run_v is None:
            run_v = dist
            run_i = jnp.zeros(dist.shape, jnp.int32)
        else:
            lt = dist < run_v
            run_v = jnp.where(lt, dist, run_v)
            run_i = jnp.where(lt, jnp.int32(j), run_i)
    lane = jax.lax.broadcasted_iota(jnp.int32, (bn, _CH), 1)
    kfull = run_i * _CH + lane
    m = jnp.min(run_v, axis=1, keepdims=True)
    cand = jnp.where(run_v == m, kfull, jnp.int32(k))
    o_ref[...] = jnp.min(cand, axis=1)


def kernel(x, centroids):
    n, d = x.shape
    k, _ = centroids.shape
    grid = (n // _BN,)
    return pl.pallas_call(
        _vq_argmin_kernel,
        grid=grid,
        in_specs=[
            pl.BlockSpec((_BN, d), lambda i: (i, 0)),
            pl.BlockSpec((k, d), lambda i: (0, 0)),
        ],
        out_specs=pl.BlockSpec((_BN,), lambda i: (i,)),
        out_shape=jax.ShapeDtypeStruct((n,), jnp.int32),
        compiler_params=pltpu.CompilerParams(
            dimension_semantics=("parallel",),
        ),
    )(x, centroids)
